# Initial kernel scaffold; baseline (speedup 1.0000x reference)
import jax
import jax.numpy as jnp
from jax import lax
from jax.experimental import pallas as pl
from jax.experimental.pallas import tpu as pltpu
from jax.experimental.pallas import tpu_sc as plsc

N_NODES = 10000
N_EDGES = 320000
EMB = 128
NUM_EMBEDDINGS = 95

NC, NS, LANES = 2, 16, 16
NW = NC * NS
EPW = N_EDGES // NW
NV = EPW // LANES

BLK = 2560
NB = N_EDGES // BLK


def _sc_gather_body(z_hbm, ii_hbm, jj_hbm, zi_hbm, zj_hbm,
                    z_v, ii_v, jj_v, oi_v, oj_v):
    wid = lax.axis_index("s") * NC + lax.axis_index("c")
    base = wid * EPW
    pltpu.sync_copy(z_hbm, z_v)
    pltpu.sync_copy(ii_hbm.at[pl.ds(base, EPW)], ii_v)
    pltpu.sync_copy(jj_hbm.at[pl.ds(base, EPW)], jj_v)

    def body(k, c):
        s = pl.ds(k * LANES, LANES)
        oi_v[s] = plsc.load_gather(z_v, [ii_v[s]])
        oj_v[s] = plsc.load_gather(z_v, [jj_v[s]])
        return c

    lax.fori_loop(0, NV, body, 0)
    pltpu.sync_copy(oi_v, zi_hbm.at[pl.ds(base, EPW)])
    pltpu.sync_copy(oj_v, zj_hbm.at[pl.ds(base, EPW)])


def _sc_gather(Z, ii, jj):
    mesh = plsc.VectorSubcoreMesh(core_axis_name="c", subcore_axis_name="s")
    f = pl.kernel(
        _sc_gather_body,
        mesh=mesh,
        out_type=(jax.ShapeDtypeStruct((N_EDGES,), jnp.int32),
                  jax.ShapeDtypeStruct((N_EDGES,), jnp.int32)),
        scratch_types=[
            pltpu.VMEM((N_NODES,), jnp.int32),
            pltpu.VMEM((EPW,), jnp.int32),
            pltpu.VMEM((EPW,), jnp.int32),
            pltpu.VMEM((EPW,), jnp.int32),
            pltpu.VMEM((EPW,), jnp.int32),
        ],
    )
    return f(Z, ii, jj)


def _silu(x):
    return x / (1.0 + jnp.exp(-x))


def _tc_body(rbf_ref, zi_ref, zj_ref, e_ref, w1_ref, w2_ref, w3_ref,
             wr_ref, br_ref, bd_ref, out_ref, t_scr):
    @pl.when(pl.program_id(0) == 0)
    def _():
        ew = e_ref[...]
        t_scr[0:128, :] = jnp.dot(ew, w1_ref[...],
                                  preferred_element_type=jnp.float32)
        t_scr[128:256, :] = jnp.dot(ew, w2_ref[...],
                                    preferred_element_type=jnp.float32)

    zi = zi_ref[0, 0, :]
    zj = zj_ref[0, 0, :]
    ci = lax.broadcasted_iota(jnp.int32, (BLK, 256), 1)
    oh = jnp.where((ci == zi[:, None]) | (ci == zj[:, None] + 128), 1.0, 0.0)
    acc = jnp.dot(oh, t_scr[...], preferred_element_type=jnp.float32)
    r = jnp.dot(rbf_ref[...], wr_ref[...],
                preferred_element_type=jnp.float32) + br_ref[...]
    r = _silu(r)
    acc = acc + jnp.dot(r, w3_ref[...],
                        preferred_element_type=jnp.float32) + bd_ref[...]
    out_ref[...] = _silu(acc)


def kernel(Z, rbf, idnb_i, idnb_j, embeddings, W_rbf, b_rbf, W_dense, b_dense):
    Z = Z.astype(jnp.int32)
    ii = idnb_i.astype(jnp.int32)
    jj = idnb_j.astype(jnp.int32)

    zi, zj = _sc_gather(Z, ii, jj)

    zi3 = zi.reshape(NB, 1, BLK)
    zj3 = zj.reshape(NB, 1, BLK)
    rbf8 = jnp.pad(rbf, ((0, 0), (0, 2)))
    epad = jnp.pad(embeddings, ((0, 128 - NUM_EMBEDDINGS), (0, 0)))
    w1 = W_dense[0:128]
    w2 = W_dense[128:256]
    w3 = W_dense[256:384]
    wr = jnp.pad(W_rbf, ((0, 2), (0, 0)))
    br = b_rbf.reshape(1, EMB)
    bd = b_dense.reshape(1, EMB)

    return pl.pallas_call(
        _tc_body,
        grid=(NB,),
        in_specs=[
            pl.BlockSpec((BLK, 8), lambda i: (i, 0)),
            pl.BlockSpec((1, 1, BLK), lambda i: (i, 0, 0)),
            pl.BlockSpec((1, 1, BLK), lambda i: (i, 0, 0)),
            pl.BlockSpec((128, 128), lambda i: (0, 0)),
            pl.BlockSpec((128, 128), lambda i: (0, 0)),
            pl.BlockSpec((128, 128), lambda i: (0, 0)),
            pl.BlockSpec((128, 128), lambda i: (0, 0)),
            pl.BlockSpec((8, 128), lambda i: (0, 0)),
            pl.BlockSpec((1, 128), lambda i: (0, 0)),
            pl.BlockSpec((1, 128), lambda i: (0, 0)),
        ],
        out_specs=pl.BlockSpec((BLK, EMB), lambda i: (i, 0)),
        out_shape=jax.ShapeDtypeStruct((N_EDGES, EMB), jnp.float32),
        scratch_shapes=[pltpu.VMEM((256, EMB), jnp.float32)],
        compiler_params=pltpu.CompilerParams(
            dimension_semantics=("arbitrary",)),
    )(rbf8, zi3, zj3, epad, w1, w2, w3, wr, br, bd)


# speedup vs baseline: 14.3015x; 14.3015x over previous
import jax
import jax.numpy as jnp
from jax import lax
from jax.experimental import pallas as pl
from jax.experimental.pallas import tpu as pltpu
from jax.experimental.pallas import tpu_sc as plsc

N_NODES = 10000
N_EDGES = 320000
EMB = 128
NUM_EMBEDDINGS = 95

NC, NS, LANES = 2, 16, 16
NW = NC * NS
EPW = N_EDGES // NW
NV = EPW // LANES

BLK = 2560
NB = N_EDGES // BLK


def _sc_gather_body(z_hbm, ii_hbm, jj_hbm, zi_hbm, zj_hbm,
                    z_v, ii_v, jj_v, oi_v, oj_v):
    wid = lax.axis_index("s") * NC + lax.axis_index("c")
    base = wid * EPW
    pltpu.sync_copy(z_hbm, z_v)
    pltpu.sync_copy(ii_hbm.at[pl.ds(base, EPW)], ii_v)
    pltpu.sync_copy(jj_hbm.at[pl.ds(base, EPW)], jj_v)

    def body(k, c):
        s = pl.ds(k * LANES, LANES)
        oi_v[s] = plsc.load_gather(z_v, [ii_v[s]])
        oj_v[s] = plsc.load_gather(z_v, [jj_v[s]])
        return c

    lax.fori_loop(0, NV, body, 0)
    pltpu.sync_copy(oi_v, zi_hbm.at[pl.ds(base, EPW)])
    pltpu.sync_copy(oj_v, zj_hbm.at[pl.ds(base, EPW)])


def _sc_gather(Z, ii, jj):
    mesh = plsc.VectorSubcoreMesh(core_axis_name="c", subcore_axis_name="s")
    f = pl.kernel(
        _sc_gather_body,
        mesh=mesh,
        out_type=(jax.ShapeDtypeStruct((N_EDGES,), jnp.int32),
                  jax.ShapeDtypeStruct((N_EDGES,), jnp.int32)),
        scratch_types=[
            pltpu.VMEM((N_NODES,), jnp.int32),
            pltpu.VMEM((EPW,), jnp.int32),
            pltpu.VMEM((EPW,), jnp.int32),
            pltpu.VMEM((EPW,), jnp.int32),
            pltpu.VMEM((EPW,), jnp.int32),
        ],
        compiler_params=pltpu.CompilerParams(needs_layout_passes=False),
    )
    return f(Z, ii, jj)


def _silu(x):
    return x / (1.0 + jnp.exp(-x))


def _tc_body(rbf_ref, zi_ref, zj_ref, e_ref, w1_ref, w2_ref, w3_ref,
             wr_ref, br_ref, bd_ref, out_ref, t_scr):
    @pl.when(pl.program_id(0) == 0)
    def _():
        ew = e_ref[...]
        t_scr[0:128, :] = jnp.dot(ew, w1_ref[...],
                                  preferred_element_type=jnp.float32)
        t_scr[128:256, :] = jnp.dot(ew, w2_ref[...],
                                    preferred_element_type=jnp.float32)

    zi = zi_ref[0, 0, :]
    zj = zj_ref[0, 0, :]
    ci = lax.broadcasted_iota(jnp.int32, (BLK, 256), 1)
    oh = jnp.where((ci == zi[:, None]) | (ci == zj[:, None] + 128), 1.0, 0.0)
    acc = jnp.dot(oh, t_scr[...], preferred_element_type=jnp.float32)
    r = jnp.dot(rbf_ref[...], wr_ref[...],
                preferred_element_type=jnp.float32) + br_ref[...]
    r = _silu(r)
    acc = acc + jnp.dot(r, w3_ref[...],
                        preferred_element_type=jnp.float32) + bd_ref[...]
    out_ref[...] = _silu(acc)


def kernel(Z, rbf, idnb_i, idnb_j, embeddings, W_rbf, b_rbf, W_dense, b_dense):
    Z = Z.astype(jnp.int32)
    ii = idnb_i.astype(jnp.int32)
    jj = idnb_j.astype(jnp.int32)

    zi, zj = _sc_gather(Z, ii, jj)

    zi3 = zi.reshape(NB, 1, BLK)
    zj3 = zj.reshape(NB, 1, BLK)
    rbf8 = jnp.pad(rbf, ((0, 0), (0, 2)))
    epad = jnp.pad(embeddings, ((0, 128 - NUM_EMBEDDINGS), (0, 0)))
    w1 = W_dense[0:128]
    w2 = W_dense[128:256]
    w3 = W_dense[256:384]
    wr = jnp.pad(W_rbf, ((0, 2), (0, 0)))
    br = b_rbf.reshape(1, EMB)
    bd = b_dense.reshape(1, EMB)

    return pl.pallas_call(
        _tc_body,
        grid=(NB,),
        in_specs=[
            pl.BlockSpec((BLK, 8), lambda i: (i, 0)),
            pl.BlockSpec((1, 1, BLK), lambda i: (i, 0, 0)),
            pl.BlockSpec((1, 1, BLK), lambda i: (i, 0, 0)),
            pl.BlockSpec((128, 128), lambda i: (0, 0)),
            pl.BlockSpec((128, 128), lambda i: (0, 0)),
            pl.BlockSpec((128, 128), lambda i: (0, 0)),
            pl.BlockSpec((128, 128), lambda i: (0, 0)),
            pl.BlockSpec((8, 128), lambda i: (0, 0)),
            pl.BlockSpec((1, 128), lambda i: (0, 0)),
            pl.BlockSpec((1, 128), lambda i: (0, 0)),
        ],
        out_specs=pl.BlockSpec((BLK, EMB), lambda i: (i, 0)),
        out_shape=jax.ShapeDtypeStruct((N_EDGES, EMB), jnp.float32),
        scratch_shapes=[pltpu.VMEM((256, EMB), jnp.float32)],
        compiler_params=pltpu.CompilerParams(
            dimension_semantics=("arbitrary",)),
    )(rbf8, zi3, zj3, epad, w1, w2, w3, wr, br, bd)


# speedup vs baseline: 15.5567x; 1.0878x over previous
import jax
import jax.numpy as jnp
from jax import lax
from jax.experimental import pallas as pl
from jax.experimental.pallas import tpu as pltpu
from jax.experimental.pallas import tpu_sc as plsc

N_NODES = 10000
N_EDGES = 320000
EMB = 128
NUM_EMBEDDINGS = 95

NC, NS, LANES = 2, 16, 16
NW = NC * NS
EPW = N_EDGES // NW
NV = EPW // LANES

BLK = 2000
NB = N_EDGES // BLK
BPW = EPW // BLK


def _sc_gather_body(z_hbm, ii_hbm, jj_hbm, zi_hbm, zj_hbm,
                    z_v, ii_v, jj_v, oi_v, oj_v, sem):
    wid = lax.axis_index("s") * NC + lax.axis_index("c")
    base = wid * EPW
    cz = pltpu.async_copy(z_hbm, z_v, sem)
    ci = pltpu.async_copy(ii_hbm.at[pl.ds(base, EPW)], ii_v, sem)
    cj = pltpu.async_copy(jj_hbm.at[pl.ds(base, EPW)], jj_v, sem)
    cz.wait()
    ci.wait()
    cj.wait()

    def body(k, c):
        for u in range(8):
            s = pl.ds((k * 8 + u) * LANES, LANES)
            oi_v[s] = plsc.load_gather(z_v, [ii_v[s]])
            oj_v[s] = plsc.load_gather(z_v, [jj_v[s]])
        return c

    lax.fori_loop(0, NV // 8, body, 0)
    for b in range(BPW):
        pltpu.sync_copy(oi_v.at[pl.ds(b * BLK, BLK)],
                        zi_hbm.at[wid * BPW + b, 0])
        pltpu.sync_copy(oj_v.at[pl.ds(b * BLK, BLK)],
                        zj_hbm.at[wid * BPW + b, 0])


def _sc_gather(Z, ii, jj):
    mesh = plsc.VectorSubcoreMesh(core_axis_name="c", subcore_axis_name="s")
    f = pl.kernel(
        _sc_gather_body,
        mesh=mesh,
        out_type=(jax.ShapeDtypeStruct((NB, 1, BLK), jnp.int32),
                  jax.ShapeDtypeStruct((NB, 1, BLK), jnp.int32)),
        scratch_types=[
            pltpu.VMEM((N_NODES,), jnp.int32),
            pltpu.VMEM((EPW,), jnp.int32),
            pltpu.VMEM((EPW,), jnp.int32),
            pltpu.VMEM((EPW,), jnp.int32),
            pltpu.VMEM((EPW,), jnp.int32),
            pltpu.SemaphoreType.DMA,
        ],
        compiler_params=pltpu.CompilerParams(needs_layout_passes=False),
    )
    return f(Z, ii, jj)


def _silu(x):
    return x / (1.0 + jnp.exp(-x))


def _tc_body(rbf_ref, zi_ref, zj_ref, e_ref, w1_ref, w2_ref, w3_ref,
             wr_ref, br_ref, bd_ref, out_ref, t_scr):
    @pl.when(pl.program_id(0) == 0)
    def _():
        ew = e_ref[...]
        t_scr[0:128, :] = jnp.dot(ew, w1_ref[...],
                                  preferred_element_type=jnp.float32)
        t_scr[128:256, :] = jnp.dot(ew, w2_ref[...],
                                    preferred_element_type=jnp.float32)

    zi = zi_ref[0, 0, :]
    zj = zj_ref[0, 0, :]
    ci = lax.broadcasted_iota(jnp.int32, (BLK, 256), 1)
    oh = jnp.where((ci == zi[:, None]) | (ci == zj[:, None] + 128), 1.0, 0.0)
    acc = jnp.dot(oh, t_scr[...], preferred_element_type=jnp.float32)
    r = jnp.dot(rbf_ref[...], wr_ref[...],
                preferred_element_type=jnp.float32) + br_ref[...]
    r = _silu(r)
    acc = acc + jnp.dot(r, w3_ref[...],
                        preferred_element_type=jnp.float32) + bd_ref[...]
    out_ref[...] = _silu(acc)


def kernel(Z, rbf, idnb_i, idnb_j, embeddings, W_rbf, b_rbf, W_dense, b_dense):
    Z = Z.astype(jnp.int32)
    ii = idnb_i.astype(jnp.int32)
    jj = idnb_j.astype(jnp.int32)

    zi3, zj3 = _sc_gather(Z, ii, jj)

    epad = jnp.pad(embeddings, ((0, 128 - NUM_EMBEDDINGS), (0, 0)))
    w1 = W_dense[0:128]
    w2 = W_dense[128:256]
    w3 = W_dense[256:384]
    br = b_rbf.reshape(1, EMB)
    bd = b_dense.reshape(1, EMB)

    return pl.pallas_call(
        _tc_body,
        grid=(NB,),
        in_specs=[
            pl.BlockSpec((BLK, 6), lambda i: (i, 0)),
            pl.BlockSpec((1, 1, BLK), lambda i: (i, 0, 0)),
            pl.BlockSpec((1, 1, BLK), lambda i: (i, 0, 0)),
            pl.BlockSpec((128, 128), lambda i: (0, 0)),
            pl.BlockSpec((128, 128), lambda i: (0, 0)),
            pl.BlockSpec((128, 128), lambda i: (0, 0)),
            pl.BlockSpec((128, 128), lambda i: (0, 0)),
            pl.BlockSpec((6, 128), lambda i: (0, 0)),
            pl.BlockSpec((1, 128), lambda i: (0, 0)),
            pl.BlockSpec((1, 128), lambda i: (0, 0)),
        ],
        out_specs=pl.BlockSpec((BLK, EMB), lambda i: (i, 0)),
        out_shape=jax.ShapeDtypeStruct((N_EDGES, EMB), jnp.float32),
        scratch_shapes=[pltpu.VMEM((256, EMB), jnp.float32)],
        compiler_params=pltpu.CompilerParams(
            dimension_semantics=("arbitrary",)),
    )(rbf, zi3, zj3, epad, w1, w2, w3, W_rbf, br, bd)
